# Initial kernel scaffold; baseline (speedup 1.0000x reference)
import jax
import jax.numpy as jnp
from jax.experimental import pallas as pl


def kernel(move_indices, table):
    raise NotImplementedError("write your pallas kernel here")


# speedup vs baseline: 7.4658x; 7.4658x over previous
import functools

import jax
import jax.numpy as jnp
from jax import lax
from jax.experimental import pallas as pl
from jax.experimental.pallas import tpu as pltpu
from jax.experimental.pallas import tpu_sc as plsc

_D = 128
_B = 4096
_H = 200
_TOTAL = _B * _H
_NC = 2
_NS = 16
_NW = _NC * _NS
_BPW = _TOTAL // _NW
_CHUNK = 128
_NCHUNK = _BPW // _CHUNK
_NBUF = 4
_NGRP = _NCHUNK // _NBUF


def _body(idx_hbm, table_hbm, out_hbm, idx_v, rows_v, *sems):
    gsems = sems[:_NBUF]
    osems = sems[_NBUF:]
    wid = lax.axis_index("s") * _NC + lax.axis_index("c")
    base = wid * _BPW

    pltpu.sync_copy(idx_hbm.at[wid], idx_v)

    def start_gather(b, g):
        pltpu.async_copy(table_hbm.at[idx_v.at[g]], rows_v.at[b], gsems[b])

    def wait_gather(b):
        pltpu.make_async_copy(table_hbm.at[idx_v.at[0]], rows_v.at[b],
                              gsems[b]).wait()

    def start_put(b, g):
        pltpu.async_copy(rows_v.at[b],
                         out_hbm.at[pl.ds(base + g * _CHUNK, _CHUNK)],
                         osems[b])

    def wait_put(b):
        pltpu.make_async_copy(rows_v.at[b],
                              out_hbm.at[pl.ds(base, _CHUNK)],
                              osems[b]).wait()

    for b in range(_NBUF):
        start_gather(b, b)

    @pl.loop(0, _NGRP - 1)
    def _grp(i):
        for b in range(_NBUF):
            wait_gather(b)
            start_put(b, i * _NBUF + b)
        for b in range(_NBUF):
            wait_put(b)
            start_gather(b, (i + 1) * _NBUF + b)

    for b in range(_NBUF):
        wait_gather(b)
        start_put(b, (_NGRP - 1) * _NBUF + b)
    for b in range(_NBUF):
        wait_put(b)


_mesh = plsc.VectorSubcoreMesh(core_axis_name="c", subcore_axis_name="s")

_gather_call = functools.partial(
    pl.kernel,
    out_type=jax.ShapeDtypeStruct((_TOTAL, _D), jnp.float32),
    mesh=_mesh,
    scratch_types=[
        pltpu.VMEM((_NCHUNK, _CHUNK), jnp.int32),
        pltpu.VMEM((_NBUF, _CHUNK, _D), jnp.float32),
    ] + [pltpu.SemaphoreType.DMA] * (2 * _NBUF),
)(_body)


@jax.jit
def kernel(move_indices, table):
    idx = move_indices.reshape(_NW, _NCHUNK, _CHUNK).astype(jnp.int32)
    out = _gather_call(idx, table)
    return out.reshape(_B, _H, _D)


# speedup vs baseline: 15.6512x; 2.0964x over previous
import functools

import jax
import jax.numpy as jnp
from jax import lax
from jax.experimental import pallas as pl
from jax.experimental.pallas import tpu as pltpu
from jax.experimental.pallas import tpu_sc as plsc

_D = 128
_B = 4096
_H = 200
_TOTAL = _B * _H
_NC = 2
_NS = 16
_NW = _NC * _NS
_BPW = _TOTAL // _NW
_CHUNK = 128
_NCHUNK = _BPW // _CHUNK
_NBUF = 4
_NGRP = _NCHUNK // _NBUF


def _body(idx_hbm, table_hbm, out_hbm, idx_v, rows_v, table_sh, *sems):
    gsems = sems[:_NBUF]
    osems = sems[_NBUF:]
    sid = lax.axis_index("s")
    wid = sid * _NC + lax.axis_index("c")
    base = wid * _BPW

    @pl.when(sid == 0)
    def _():
        pltpu.sync_copy(table_hbm, table_sh)

    pltpu.sync_copy(idx_hbm.at[wid], idx_v)
    plsc.subcore_barrier()

    def start_gather(b, g):
        pltpu.async_copy(table_sh.at[idx_v.at[g]], rows_v.at[b], gsems[b])

    def wait_gather(b):
        pltpu.make_async_copy(table_sh.at[idx_v.at[0]], rows_v.at[b],
                              gsems[b]).wait()

    def start_put(b, g):
        pltpu.async_copy(rows_v.at[b],
                         out_hbm.at[pl.ds(base + g * _CHUNK, _CHUNK)],
                         osems[b])

    def wait_put(b):
        pltpu.make_async_copy(rows_v.at[b],
                              out_hbm.at[pl.ds(base, _CHUNK)],
                              osems[b]).wait()

    for b in range(_NBUF):
        start_gather(b, b)

    @pl.loop(0, _NGRP - 1)
    def _grp(i):
        for b in range(_NBUF):
            wait_gather(b)
            start_put(b, i * _NBUF + b)
        for b in range(_NBUF):
            wait_put(b)
            start_gather(b, (i + 1) * _NBUF + b)

    for b in range(_NBUF):
        wait_gather(b)
        start_put(b, (_NGRP - 1) * _NBUF + b)
    for b in range(_NBUF):
        wait_put(b)


_mesh = plsc.VectorSubcoreMesh(core_axis_name="c", subcore_axis_name="s")

_gather_call = functools.partial(
    pl.kernel,
    out_type=jax.ShapeDtypeStruct((_TOTAL, _D), jnp.float32),
    mesh=_mesh,
    scratch_types=[
        pltpu.VMEM((_NCHUNK, _CHUNK), jnp.int32),
        pltpu.VMEM((_NBUF, _CHUNK, _D), jnp.float32),
        pltpu.VMEM_SHARED((1858, _D), jnp.float32),
    ] + [pltpu.SemaphoreType.DMA] * (2 * _NBUF),
)(_body)


@jax.jit
def kernel(move_indices, table):
    idx = move_indices.reshape(_NW, _NCHUNK, _CHUNK).astype(jnp.int32)
    out = _gather_call(idx, table)
    return out.reshape(_B, _H, _D)
